# baseline (speedup 1.0000x reference)
import dataclasses
import functools

import jax
import jax.numpy as jnp
from jax import lax
from jax.experimental import pallas as pl
from jax.experimental.pallas import tpu as pltpu
from jax.experimental.pallas import tpu_sc as plsc

N = 10000
E = 320000
D = 128
ED = 16
K = 3

NC = 2
NS = 16
NW = NC * NS
PER_W = E // NW
C = 200
NCHUNK = PER_W // C
NPAD = 10240
ROWS_PER_SUB = NPAD // NS
ZR = 64

_f32 = jnp.float32
_mesh = plsc.VectorSubcoreMesh(core_axis_name="c", subcore_axis_name="s")

_sc_params = pltpu.CompilerParams()
if "needs_layout_passes" in pltpu.CompilerParams.__dataclass_fields__:
    _sc_params = dataclasses.replace(_sc_params, needs_layout_passes=False)


def _s1_body(h_hbm, src_hbm, dst_hbm, g_hbm,
             idx_s0, idx_d0, idx_s1, idx_d1,
             rows_s0, rows_d0, rows_s1, rows_d1, sem0, sem1):
    cid = lax.axis_index("c")
    sid = lax.axis_index("s")
    base0 = (sid * NC + cid) * PER_W

    def fetch(ci, idx_s, idx_d, rows_s, rows_d, sem):
        base = base0 + ci * C
        pltpu.sync_copy(src_hbm.at[pl.ds(base, C)], idx_s)
        pltpu.sync_copy(dst_hbm.at[pl.ds(base, C)], idx_d)
        pltpu.async_copy(h_hbm.at[idx_s], rows_s, sem)
        pltpu.async_copy(h_hbm.at[idx_d], rows_d, sem)

    def drain(rows_s, rows_d, sem):
        pltpu.make_async_copy(h_hbm.at[pl.ds(0, C)], rows_s, sem).wait()
        pltpu.make_async_copy(h_hbm.at[pl.ds(0, C)], rows_d, sem).wait()

    def process(ci, rows_s, rows_d):
        @pl.loop(0, C)
        def _edge(e):
            for r in range(D // 16):
                sl = pl.ds(r * 16, 16)
                rows_s.at[e, sl][...] = (rows_s.at[e, sl][...] +
                                         rows_d.at[e, sl][...])

        pltpu.sync_copy(rows_s, g_hbm.at[pl.ds(base0 + ci * C, C)])

    fetch(0, idx_s0, idx_d0, rows_s0, rows_d0, sem0)

    @pl.loop(0, NCHUNK, step=2)
    def _chunk(ci):
        fetch(ci + 1, idx_s1, idx_d1, rows_s1, rows_d1, sem1)
        drain(rows_s0, rows_d0, sem0)
        process(ci, rows_s0, rows_d0)

        @pl.when(ci + 2 < NCHUNK)
        def _pre():
            fetch(ci + 2, idx_s0, idx_d0, rows_s0, rows_d0, sem0)

        drain(rows_s1, rows_d1, sem1)
        process(ci + 1, rows_s1, rows_d1)


_s1 = functools.partial(
    pl.kernel,
    out_type=jax.ShapeDtypeStruct((E, D), _f32),
    mesh=_mesh,
    scratch_types=[
        pltpu.VMEM((C,), jnp.int32),
        pltpu.VMEM((C,), jnp.int32),
        pltpu.VMEM((C,), jnp.int32),
        pltpu.VMEM((C,), jnp.int32),
        pltpu.VMEM((C, D), _f32),
        pltpu.VMEM((C, D), _f32),
        pltpu.VMEM((C, D), _f32),
        pltpu.VMEM((C, D), _f32),
        pltpu.SemaphoreType.DMA,
        pltpu.SemaphoreType.DMA,
    ],
)(_s1_body)


NH = NPAD // 2
NHT = NH + 16
C2 = 80
PER_SUB = E // NS
NCHUNK2 = PER_SUB // C2
ROWS_PER_SUB2 = NH // NS
DR = NH // 8


def _s2_body(h_hbm, src_hbm, dst_hbm, dend_hbm, ex_hbm, zn_hbm,
             nump_hbm, denp_hbm,
             idx_s0, idx_d0, idx_dd0, exs0,
             idx_s1, idx_d1, idx_dd1, exs1,
             rows0, rows1, denrows, num_acc, den_acc, semg0, semg1):
    cid = lax.axis_index("c")
    sid = lax.axis_index("s")
    row0 = sid * ROWS_PER_SUB2

    pltpu.sync_copy(zn_hbm.at[pl.ds(row0, ROWS_PER_SUB2)],
                    num_acc.at[pl.ds(row0, ROWS_PER_SUB2)])

    @pl.when(sid == 0)
    def _zgar():
        pltpu.sync_copy(zn_hbm.at[pl.ds(NH, 16)], num_acc.at[pl.ds(NH, 16)])
        pltpu.sync_copy(zn_hbm.at[pl.ds(0, DR + 8)], den_acc)

    plsc.subcore_barrier()

    base0 = sid * PER_SUB

    def fetch(ci, idx_s, idx_d, idx_dd, exs, rows, sem):
        base = base0 + ci * C2
        pltpu.sync_copy(src_hbm.at[pl.ds(base, C2)], idx_s)
        pltpu.sync_copy(dst_hbm.at[pl.ds(cid * E + base, C2)], idx_d)
        pltpu.sync_copy(dend_hbm.at[pl.ds(cid * E + base, C2)], idx_dd)
        pltpu.sync_copy(ex_hbm.at[pl.ds(base, C2)], exs)
        pltpu.async_copy(h_hbm.at[idx_s], rows, sem)

    def process(rows, idx_d, idx_dd, exs, sem):
        pltpu.make_async_copy(h_hbm.at[pl.ds(0, C2)], rows, sem).wait()

        @pl.loop(0, C2 // 16)
        def _grp(g):
            exv = exs[pl.ds(g * 16, 16)]
            iv = idx_d[pl.ds(g * 16, 16)]
            zv = jnp.zeros((16,), _f32)
            for j in range(16):
                e = g * 16 + j
                ev = jnp.full((16,), exv[j], _f32)
                for r in range(D // 16):
                    sl = pl.ds(r * 16, 16)
                    rows.at[e, sl][...] = rows.at[e, sl][...] * ev
                for r in range(D // 16):
                    denrows.at[e, pl.ds(r * 16, 16)][...] = zv
                off = (iv[j] & 7) * 16
                denrows.at[e, pl.ds(off, 16)][...] = ev

        pltpu.sync_copy(rows, num_acc.at[idx_d], add=True)
        pltpu.sync_copy(denrows, den_acc.at[idx_dd], add=True)

    fetch(0, idx_s0, idx_d0, idx_dd0, exs0, rows0, semg0)

    @pl.loop(0, NCHUNK2, step=2)
    def _chunk(ci):
        fetch(ci + 1, idx_s1, idx_d1, idx_dd1, exs1, rows1, semg1)
        process(rows0, idx_d0, idx_dd0, exs0, semg0)

        @pl.when(ci + 2 < NCHUNK2)
        def _pre():
            fetch(ci + 2, idx_s0, idx_d0, idx_dd0, exs0, rows0, semg0)

        process(rows1, idx_d1, idx_dd1, exs1, semg1)

    plsc.subcore_barrier()
    out0 = cid * NH + row0
    pltpu.sync_copy(num_acc.at[pl.ds(row0, ROWS_PER_SUB2)],
                    nump_hbm.at[pl.ds(out0, ROWS_PER_SUB2)])
    dpw = DR // NS
    pltpu.sync_copy(den_acc.at[pl.ds(sid * dpw, dpw)],
                    denp_hbm.at[pl.ds(cid * DR + sid * dpw, dpw)])


_s2 = functools.partial(
    pl.kernel,
    out_type=(jax.ShapeDtypeStruct((NPAD, D), _f32),
              jax.ShapeDtypeStruct((NC * DR, D), _f32)),
    mesh=_mesh,
    scratch_types=[
        pltpu.VMEM((C2,), jnp.int32),
        pltpu.VMEM((C2,), jnp.int32),
        pltpu.VMEM((C2,), jnp.int32),
        pltpu.VMEM((C2,), _f32),
        pltpu.VMEM((C2,), jnp.int32),
        pltpu.VMEM((C2,), jnp.int32),
        pltpu.VMEM((C2,), jnp.int32),
        pltpu.VMEM((C2,), _f32),
        pltpu.VMEM((C2, D), _f32),
        pltpu.VMEM((C2, D), _f32),
        pltpu.VMEM((C2, D), _f32),
        pltpu.VMEM_SHARED((NHT, D), _f32),
        pltpu.VMEM_SHARED((DR + 8, D), _f32),
        pltpu.SemaphoreType.DMA,
        pltpu.SemaphoreType.DMA,
    ],
)(_s2_body)


def _t0_body(d_ref, o_ref, od_ref):
    c = pl.program_id(0)
    d = d_ref[...]
    local = d - c * NH
    ok = (local >= 0) & (local < NH)
    lane = jax.lax.broadcasted_iota(jnp.int32, d.shape, 1)
    o_ref[...] = jnp.where(ok, local, NH + (lane % 16))[None]
    od_ref[...] = jnp.where(ok, local >> 3, DR + (lane % 8))[None]


def _t0(dst2d):
    nbe, be = dst2d.shape
    return pl.pallas_call(
        _t0_body,
        grid=(NC,),
        in_specs=[pl.BlockSpec((nbe, be), lambda c: (0, 0))],
        out_specs=[pl.BlockSpec((1, nbe, be), lambda c: (c, 0, 0)),
                   pl.BlockSpec((1, nbe, be), lambda c: (c, 0, 0))],
        out_shape=(jax.ShapeDtypeStruct((NC, nbe, be), jnp.int32),
                   jax.ShapeDtypeStruct((NC, nbe, be), jnp.int32)),
    )(dst2d)


def _t1_body(x_ref, w_ref, h_ref):
    h_ref[0] = lax.dot_general(
        x_ref[0], w_ref[0], (((1,), (0,)), ((), ())),
        precision=lax.Precision.HIGHEST, preferred_element_type=_f32)


def _t1(xT, W, bn=2000):
    return pl.pallas_call(
        _t1_body,
        grid=(K, N // bn),
        in_specs=[pl.BlockSpec((1, bn, D), lambda k, i: (k, i, 0)),
                  pl.BlockSpec((1, D, D), lambda k, i: (k, 0, 0))],
        out_specs=pl.BlockSpec((1, bn, D), lambda k, i: (k, i, 0)),
        out_shape=jax.ShapeDtypeStruct((K, N, D), _f32),
    )(xT, W)


def _t2_body(g_ref, ea_ref, we_ref, a_ref, lg_ref, ex_ref):
    ef = lax.dot_general(ea_ref[...], we_ref[...], (((1,), (0,)), ((), ())),
                         precision=lax.Precision.HIGHEST,
                         preferred_element_type=_f32)
    m = g_ref[...] + ef
    l = jnp.where(m >= 0, m, 0.01 * m)
    logit = jnp.sum(l * a_ref[...], axis=1)
    lg_ref[...] = logit[None, None]
    ex_ref[...] = jnp.exp(logit)[None, None]


def _t2(g, edge_attr, We_k, a_k, be=2560):
    nbe = E // be
    return pl.pallas_call(
        _t2_body,
        grid=(nbe,),
        in_specs=[pl.BlockSpec((be, D), lambda i: (i, 0)),
                  pl.BlockSpec((be, ED), lambda i: (i, 0)),
                  pl.BlockSpec((ED, D), lambda i: (0, 0)),
                  pl.BlockSpec((1, D), lambda i: (0, 0))],
        out_specs=[pl.BlockSpec((1, 1, be), lambda i: (i, 0, 0)),
                   pl.BlockSpec((1, 1, be), lambda i: (i, 0, 0))],
        out_shape=(jax.ShapeDtypeStruct((nbe, 1, be), _f32),
                   jax.ShapeDtypeStruct((nbe, 1, be), _f32)),
    )(g, edge_attr, We_k, a_k.reshape(1, D))


_T3BN = 2048


def _t3_body(np_ref, dp_ref, o_ref):
    p = dp_ref[...]
    rep = jnp.broadcast_to(p[:, None, :], (_T3BN // 8, 8, D)).reshape(_T3BN, D)
    lane = jax.lax.broadcasted_iota(jnp.int32, (_T3BN, D), 1)
    sub = jax.lax.broadcasted_iota(jnp.int32, (_T3BN, D), 0)
    sel = ((lane >> 4) == (sub & 7)).astype(_f32)
    den = jnp.sum(rep * sel, axis=1, keepdims=True) * (1.0 / 16.0)
    o_ref[...] = np_ref[...] / (den + 1e-16)


def _t3(nump, denp):
    bn = _T3BN
    return pl.pallas_call(
        _t3_body,
        grid=(NPAD // bn,),
        in_specs=[pl.BlockSpec((bn, D), lambda i: (i, 0)),
                  pl.BlockSpec((bn // 8, D), lambda i: (i, 0))],
        out_specs=pl.BlockSpec((bn, D), lambda i: (i, 0)),
        out_shape=jax.ShapeDtypeStruct((NPAD, D), _f32),
    )(nump, denp)


def kernel(x, edge_attr, edge_index, W, We, a):
    src = edge_index[0]
    dst = edge_index[1]
    xT = jnp.transpose(x, (2, 0, 1))
    h = _t1(xT, W)
    zn = jnp.zeros((NHT, D), _f32)
    dstr, dend = _t0(dst.reshape(E // 128, 128))
    dstr = dstr.reshape(NC * E)
    dend = dend.reshape(NC * E)

    outs = []
    logits = []
    for k in range(K):
        h_k = h[k]
        g = _s1(h_k, src, dst)
        lg, ex = _t2(g, edge_attr, We[k], a[k])
        ex_flat = ex.reshape(E)
        nump, denp = _s2(h_k, src, dstr, dend, ex_flat, zn)
        outs.append(_t3(nump, denp)[:N])
        logits.append(lg.reshape(E))

    node_embeddings = jnp.stack(outs, axis=-1)
    alphas = jnp.stack(logits, axis=-1)
    return (node_embeddings, alphas)


# speedup vs baseline: 1.0312x; 1.0312x over previous
import dataclasses
import functools

import jax
import jax.numpy as jnp
from jax import lax
from jax.experimental import pallas as pl
from jax.experimental.pallas import tpu as pltpu
from jax.experimental.pallas import tpu_sc as plsc

N = 10000
E = 320000
D = 128
ED = 16
K = 3

NC = 2
NS = 16
NW = NC * NS
PER_W = E // NW
C = 200
NCHUNK = PER_W // C
NPAD = 10240
ROWS_PER_SUB = NPAD // NS
ZR = 64

_f32 = jnp.float32
_mesh = plsc.VectorSubcoreMesh(core_axis_name="c", subcore_axis_name="s")

_sc_params = pltpu.CompilerParams()
if "needs_layout_passes" in pltpu.CompilerParams.__dataclass_fields__:
    _sc_params = dataclasses.replace(_sc_params, needs_layout_passes=False)


def _s1_body(h_hbm, src_hbm, dst_hbm, g_hbm,
             idx_s0, idx_d0, idx_s1, idx_d1,
             rows_s0, rows_d0, rows_s1, rows_d1, sem0, sem1):
    cid = lax.axis_index("c")
    sid = lax.axis_index("s")
    base0 = (sid * NC + cid) * PER_W

    def fetch(ci, idx_s, idx_d, rows_s, rows_d, sem):
        base = base0 + ci * C
        pltpu.sync_copy(src_hbm.at[pl.ds(base, C)], idx_s)
        pltpu.sync_copy(dst_hbm.at[pl.ds(base, C)], idx_d)
        pltpu.async_copy(h_hbm.at[idx_s], rows_s, sem)
        pltpu.async_copy(h_hbm.at[idx_d], rows_d, sem)

    def drain(rows_s, rows_d, sem):
        pltpu.make_async_copy(h_hbm.at[pl.ds(0, C)], rows_s, sem).wait()
        pltpu.make_async_copy(h_hbm.at[pl.ds(0, C)], rows_d, sem).wait()

    def process(ci, rows_s, rows_d):
        @pl.loop(0, C)
        def _edge(e):
            for r in range(D // 16):
                sl = pl.ds(r * 16, 16)
                rows_s.at[e, sl][...] = (rows_s.at[e, sl][...] +
                                         rows_d.at[e, sl][...])

        pltpu.sync_copy(rows_s, g_hbm.at[pl.ds(base0 + ci * C, C)])

    fetch(0, idx_s0, idx_d0, rows_s0, rows_d0, sem0)

    @pl.loop(0, NCHUNK, step=2)
    def _chunk(ci):
        fetch(ci + 1, idx_s1, idx_d1, rows_s1, rows_d1, sem1)
        drain(rows_s0, rows_d0, sem0)
        process(ci, rows_s0, rows_d0)

        @pl.when(ci + 2 < NCHUNK)
        def _pre():
            fetch(ci + 2, idx_s0, idx_d0, rows_s0, rows_d0, sem0)

        drain(rows_s1, rows_d1, sem1)
        process(ci + 1, rows_s1, rows_d1)


_s1 = functools.partial(
    pl.kernel,
    out_type=jax.ShapeDtypeStruct((E, D), _f32),
    mesh=_mesh,
    scratch_types=[
        pltpu.VMEM((C,), jnp.int32),
        pltpu.VMEM((C,), jnp.int32),
        pltpu.VMEM((C,), jnp.int32),
        pltpu.VMEM((C,), jnp.int32),
        pltpu.VMEM((C, D), _f32),
        pltpu.VMEM((C, D), _f32),
        pltpu.VMEM((C, D), _f32),
        pltpu.VMEM((C, D), _f32),
        pltpu.SemaphoreType.DMA,
        pltpu.SemaphoreType.DMA,
    ],
)(_s1_body)


NH = NPAD // 2
NHT = NH + 16
C2 = 160
PER_SUB = E // NS
NCHUNK2 = PER_SUB // C2
ROWS_PER_SUB2 = NH // NS
DR = NH // 8


def _s2_body(h_hbm, src_hbm, dst_hbm, dend_hbm, ex_hbm, zn_hbm,
             nump_hbm, denp_hbm,
             idx_s, idx_d, idx_dd, exs, rows, denrows, num_acc, den_acc):
    cid = lax.axis_index("c")
    sid = lax.axis_index("s")
    row0 = sid * ROWS_PER_SUB2

    pltpu.sync_copy(zn_hbm.at[pl.ds(row0, ROWS_PER_SUB2)],
                    num_acc.at[pl.ds(row0, ROWS_PER_SUB2)])

    @pl.when(sid == 0)
    def _zgar():
        pltpu.sync_copy(zn_hbm.at[pl.ds(NH, 16)], num_acc.at[pl.ds(NH, 16)])
        pltpu.sync_copy(zn_hbm.at[pl.ds(0, DR + 8)], den_acc)

    plsc.subcore_barrier()

    base0 = sid * PER_SUB

    @pl.loop(0, NCHUNK2)
    def _chunk(ci):
        base = base0 + ci * C2
        pltpu.sync_copy(src_hbm.at[pl.ds(base, C2)], idx_s)
        pltpu.sync_copy(dst_hbm.at[pl.ds(cid * E + base, C2)], idx_d)
        pltpu.sync_copy(dend_hbm.at[pl.ds(cid * E + base, C2)], idx_dd)
        pltpu.sync_copy(ex_hbm.at[pl.ds(base, C2)], exs)
        pltpu.sync_copy(h_hbm.at[idx_s], rows)

        @pl.loop(0, C2 // 16)
        def _grp(g):
            exv = exs[pl.ds(g * 16, 16)]
            iv = idx_d[pl.ds(g * 16, 16)]
            zv = jnp.zeros((16,), _f32)
            for j in range(16):
                e = g * 16 + j
                ev = jnp.full((16,), exv[j], _f32)
                for r in range(D // 16):
                    sl = pl.ds(r * 16, 16)
                    rows.at[e, sl][...] = rows.at[e, sl][...] * ev
                for r in range(D // 16):
                    denrows.at[e, pl.ds(r * 16, 16)][...] = zv
                off = (iv[j] & 7) * 16
                denrows.at[e, pl.ds(off, 16)][...] = ev

        pltpu.sync_copy(rows, num_acc.at[idx_d], add=True)
        pltpu.sync_copy(denrows, den_acc.at[idx_dd], add=True)

    plsc.subcore_barrier()
    out0 = cid * NH + row0
    pltpu.sync_copy(num_acc.at[pl.ds(row0, ROWS_PER_SUB2)],
                    nump_hbm.at[pl.ds(out0, ROWS_PER_SUB2)])
    dpw = DR // NS
    pltpu.sync_copy(den_acc.at[pl.ds(sid * dpw, dpw)],
                    denp_hbm.at[pl.ds(cid * DR + sid * dpw, dpw)])


_s2 = functools.partial(
    pl.kernel,
    out_type=(jax.ShapeDtypeStruct((NPAD, D), _f32),
              jax.ShapeDtypeStruct((NC * DR, D), _f32)),
    mesh=_mesh,
    scratch_types=[
        pltpu.VMEM((C2,), jnp.int32),
        pltpu.VMEM((C2,), jnp.int32),
        pltpu.VMEM((C2,), jnp.int32),
        pltpu.VMEM((C2,), _f32),
        pltpu.VMEM((C2, D), _f32),
        pltpu.VMEM((C2, D), _f32),
        pltpu.VMEM_SHARED((NHT, D), _f32),
        pltpu.VMEM_SHARED((DR + 8, D), _f32),
    ],
)(_s2_body)


def _t0_body(d_ref, o_ref, od_ref):
    c = pl.program_id(0)
    d = d_ref[...]
    local = d - c * NH
    ok = (local >= 0) & (local < NH)
    lane = jax.lax.broadcasted_iota(jnp.int32, d.shape, 1)
    o_ref[...] = jnp.where(ok, local, NH + (lane % 16))[None]
    od_ref[...] = jnp.where(ok, local >> 3, DR + (lane % 8))[None]


def _t0(dst2d):
    nbe, be = dst2d.shape
    return pl.pallas_call(
        _t0_body,
        grid=(NC,),
        in_specs=[pl.BlockSpec((nbe, be), lambda c: (0, 0))],
        out_specs=[pl.BlockSpec((1, nbe, be), lambda c: (c, 0, 0)),
                   pl.BlockSpec((1, nbe, be), lambda c: (c, 0, 0))],
        out_shape=(jax.ShapeDtypeStruct((NC, nbe, be), jnp.int32),
                   jax.ShapeDtypeStruct((NC, nbe, be), jnp.int32)),
    )(dst2d)


def _t1_body(x_ref, w_ref, h_ref):
    h_ref[0] = lax.dot_general(
        x_ref[0], w_ref[0], (((1,), (0,)), ((), ())),
        precision=lax.Precision.HIGHEST, preferred_element_type=_f32)


def _t1(xT, W, bn=2000):
    return pl.pallas_call(
        _t1_body,
        grid=(K, N // bn),
        in_specs=[pl.BlockSpec((1, bn, D), lambda k, i: (k, i, 0)),
                  pl.BlockSpec((1, D, D), lambda k, i: (k, 0, 0))],
        out_specs=pl.BlockSpec((1, bn, D), lambda k, i: (k, i, 0)),
        out_shape=jax.ShapeDtypeStruct((K, N, D), _f32),
    )(xT, W)


def _t2_body(g_ref, ea_ref, we_ref, a_ref, lg_ref, ex_ref):
    ef = lax.dot_general(ea_ref[...], we_ref[...], (((1,), (0,)), ((), ())),
                         precision=lax.Precision.HIGHEST,
                         preferred_element_type=_f32)
    m = g_ref[...] + ef
    l = jnp.where(m >= 0, m, 0.01 * m)
    logit = jnp.sum(l * a_ref[...], axis=1)
    lg_ref[...] = logit[None, None]
    ex_ref[...] = jnp.exp(logit)[None, None]


def _t2(g, edge_attr, We_k, a_k, be=2560):
    nbe = E // be
    return pl.pallas_call(
        _t2_body,
        grid=(nbe,),
        in_specs=[pl.BlockSpec((be, D), lambda i: (i, 0)),
                  pl.BlockSpec((be, ED), lambda i: (i, 0)),
                  pl.BlockSpec((ED, D), lambda i: (0, 0)),
                  pl.BlockSpec((1, D), lambda i: (0, 0))],
        out_specs=[pl.BlockSpec((1, 1, be), lambda i: (i, 0, 0)),
                   pl.BlockSpec((1, 1, be), lambda i: (i, 0, 0))],
        out_shape=(jax.ShapeDtypeStruct((nbe, 1, be), _f32),
                   jax.ShapeDtypeStruct((nbe, 1, be), _f32)),
    )(g, edge_attr, We_k, a_k.reshape(1, D))


_T3BN = 2048


def _t3_body(np_ref, dp_ref, o_ref):
    p = dp_ref[...]
    rep = jnp.broadcast_to(p[:, None, :], (_T3BN // 8, 8, D)).reshape(_T3BN, D)
    lane = jax.lax.broadcasted_iota(jnp.int32, (_T3BN, D), 1)
    sub = jax.lax.broadcasted_iota(jnp.int32, (_T3BN, D), 0)
    sel = ((lane >> 4) == (sub & 7)).astype(_f32)
    den = jnp.sum(rep * sel, axis=1, keepdims=True) * (1.0 / 16.0)
    o_ref[...] = np_ref[...] / (den + 1e-16)


def _t3(nump, denp):
    bn = _T3BN
    return pl.pallas_call(
        _t3_body,
        grid=(NPAD // bn,),
        in_specs=[pl.BlockSpec((bn, D), lambda i: (i, 0)),
                  pl.BlockSpec((bn // 8, D), lambda i: (i, 0))],
        out_specs=pl.BlockSpec((bn, D), lambda i: (i, 0)),
        out_shape=jax.ShapeDtypeStruct((NPAD, D), _f32),
    )(nump, denp)


def kernel(x, edge_attr, edge_index, W, We, a):
    src = edge_index[0]
    dst = edge_index[1]
    xT = jnp.transpose(x, (2, 0, 1))
    h = _t1(xT, W)
    zn = jnp.zeros((NHT, D), _f32)
    dstr, dend = _t0(dst.reshape(E // 128, 128))
    dstr = dstr.reshape(NC * E)
    dend = dend.reshape(NC * E)

    outs = []
    logits = []
    for k in range(K):
        h_k = h[k]
        g = _s1(h_k, src, dst)
        lg, ex = _t2(g, edge_attr, We[k], a[k])
        ex_flat = ex.reshape(E)
        nump, denp = _s2(h_k, src, dstr, dend, ex_flat, zn)
        outs.append(_t3(nump, denp)[:N])
        logits.append(lg.reshape(E))

    node_embeddings = jnp.stack(outs, axis=-1)
    alphas = jnp.stack(logits, axis=-1)
    return (node_embeddings, alphas)


# speedup vs baseline: 1.0320x; 1.0008x over previous
import functools

import jax
import jax.numpy as jnp
from jax import lax
from jax.experimental import pallas as pl
from jax.experimental.pallas import tpu as pltpu
from jax.experimental.pallas import tpu_sc as plsc

N = 10000
E = 320000
D = 128
ED = 16
K = 3

NC = 2
NS = 16
NW = NC * NS
PER_W = E // NW
C = 200
NCHUNK = PER_W // C
NPAD = 10240

_f32 = jnp.float32
_mesh = plsc.VectorSubcoreMesh(core_axis_name="c", subcore_axis_name="s")

def _s1_body(h_hbm, src_hbm, dst_hbm, g_hbm,
             idx_s0, idx_d0, idx_s1, idx_d1,
             rows_s0, rows_d0, rows_s1, rows_d1, sem0, sem1):
    cid = lax.axis_index("c")
    sid = lax.axis_index("s")
    base0 = (sid * NC + cid) * PER_W

    def fetch(ci, idx_s, idx_d, rows_s, rows_d, sem):
        base = base0 + ci * C
        pltpu.sync_copy(src_hbm.at[pl.ds(base, C)], idx_s)
        pltpu.sync_copy(dst_hbm.at[pl.ds(base, C)], idx_d)
        pltpu.async_copy(h_hbm.at[idx_s], rows_s, sem)
        pltpu.async_copy(h_hbm.at[idx_d], rows_d, sem)

    def drain(rows_s, rows_d, sem):
        pltpu.make_async_copy(h_hbm.at[pl.ds(0, C)], rows_s, sem).wait()
        pltpu.make_async_copy(h_hbm.at[pl.ds(0, C)], rows_d, sem).wait()

    def process(ci, rows_s, rows_d):
        @pl.loop(0, C)
        def _edge(e):
            for r in range(D // 16):
                sl = pl.ds(r * 16, 16)
                rows_s.at[e, sl][...] = (rows_s.at[e, sl][...] +
                                         rows_d.at[e, sl][...])

        pltpu.sync_copy(rows_s, g_hbm.at[pl.ds(base0 + ci * C, C)])

    fetch(0, idx_s0, idx_d0, rows_s0, rows_d0, sem0)

    @pl.loop(0, NCHUNK, step=2)
    def _chunk(ci):
        fetch(ci + 1, idx_s1, idx_d1, rows_s1, rows_d1, sem1)
        drain(rows_s0, rows_d0, sem0)
        process(ci, rows_s0, rows_d0)

        @pl.when(ci + 2 < NCHUNK)
        def _pre():
            fetch(ci + 2, idx_s0, idx_d0, rows_s0, rows_d0, sem0)

        drain(rows_s1, rows_d1, sem1)
        process(ci + 1, rows_s1, rows_d1)


_s1 = functools.partial(
    pl.kernel,
    out_type=jax.ShapeDtypeStruct((E, D), _f32),
    mesh=_mesh,
    scratch_types=[
        pltpu.VMEM((C,), jnp.int32),
        pltpu.VMEM((C,), jnp.int32),
        pltpu.VMEM((C,), jnp.int32),
        pltpu.VMEM((C,), jnp.int32),
        pltpu.VMEM((C, D), _f32),
        pltpu.VMEM((C, D), _f32),
        pltpu.VMEM((C, D), _f32),
        pltpu.VMEM((C, D), _f32),
        pltpu.SemaphoreType.DMA,
        pltpu.SemaphoreType.DMA,
    ],
)(_s1_body)


NH = NPAD // 2
NHT = NH + 16
C2 = 160
PER_SUB = E // NS
NCHUNK2 = PER_SUB // C2
ROWS_PER_SUB2 = NH // NS
DR = NH // 8


def _s2_body(h_hbm, src_hbm, dst_hbm, dend_hbm, ex_hbm, zn_hbm,
             nump_hbm, denp_hbm,
             idx_s, idx_d, idx_dd, exs, rows, denrows, num_acc, den_acc):
    cid = lax.axis_index("c")
    sid = lax.axis_index("s")
    row0 = sid * ROWS_PER_SUB2

    pltpu.sync_copy(zn_hbm.at[pl.ds(row0, ROWS_PER_SUB2)],
                    num_acc.at[pl.ds(row0, ROWS_PER_SUB2)])

    @pl.when(sid == 0)
    def _zgar():
        pltpu.sync_copy(zn_hbm.at[pl.ds(NH, 16)], num_acc.at[pl.ds(NH, 16)])
        pltpu.sync_copy(zn_hbm.at[pl.ds(0, DR + 8)], den_acc)

    plsc.subcore_barrier()

    base0 = sid * PER_SUB

    @pl.loop(0, NCHUNK2)
    def _chunk(ci):
        base = base0 + ci * C2
        pltpu.sync_copy(src_hbm.at[pl.ds(base, C2)], idx_s)
        pltpu.sync_copy(dst_hbm.at[pl.ds(cid * E + base, C2)], idx_d)
        pltpu.sync_copy(dend_hbm.at[pl.ds(cid * E + base, C2)], idx_dd)
        pltpu.sync_copy(ex_hbm.at[pl.ds(base, C2)], exs)
        pltpu.sync_copy(h_hbm.at[idx_s], rows)

        @pl.loop(0, C2 // 16)
        def _grp(g):
            exv = exs[pl.ds(g * 16, 16)]
            iv = idx_d[pl.ds(g * 16, 16)]
            zv = jnp.zeros((16,), _f32)
            for j in range(16):
                e = g * 16 + j
                ev = jnp.full((16,), exv[j], _f32)
                for r in range(D // 16):
                    sl = pl.ds(r * 16, 16)
                    rows.at[e, sl][...] = rows.at[e, sl][...] * ev
                for r in range(D // 16):
                    denrows.at[e, pl.ds(r * 16, 16)][...] = zv
                off = (iv[j] & 7) * 16
                denrows.at[e, pl.ds(off, 16)][...] = ev

        pltpu.sync_copy(rows, num_acc.at[idx_d], add=True)
        pltpu.sync_copy(denrows, den_acc.at[idx_dd], add=True)

    plsc.subcore_barrier()
    out0 = cid * NH + row0
    pltpu.sync_copy(num_acc.at[pl.ds(row0, ROWS_PER_SUB2)],
                    nump_hbm.at[pl.ds(out0, ROWS_PER_SUB2)])
    dpw = DR // NS
    pltpu.sync_copy(den_acc.at[pl.ds(sid * dpw, dpw)],
                    denp_hbm.at[pl.ds(cid * DR + sid * dpw, dpw)])


_s2 = functools.partial(
    pl.kernel,
    out_type=(jax.ShapeDtypeStruct((NPAD, D), _f32),
              jax.ShapeDtypeStruct((NC * DR, D), _f32)),
    mesh=_mesh,
    scratch_types=[
        pltpu.VMEM((C2,), jnp.int32),
        pltpu.VMEM((C2,), jnp.int32),
        pltpu.VMEM((C2,), jnp.int32),
        pltpu.VMEM((C2,), _f32),
        pltpu.VMEM((C2, D), _f32),
        pltpu.VMEM((C2, D), _f32),
        pltpu.VMEM_SHARED((NHT, D), _f32),
        pltpu.VMEM_SHARED((DR + 8, D), _f32),
    ],
)(_s2_body)


def _t0_body(d_ref, o_ref, od_ref):
    c = pl.program_id(0)
    d = d_ref[...]
    local = d - c * NH
    ok = (local >= 0) & (local < NH)
    lane = jax.lax.broadcasted_iota(jnp.int32, d.shape, 1)
    o_ref[...] = jnp.where(ok, local, NH + (lane % 16))[None]
    od_ref[...] = jnp.where(ok, local >> 3, DR + (lane % 8))[None]


def _t0(dst2d):
    nbe, be = dst2d.shape
    return pl.pallas_call(
        _t0_body,
        grid=(NC,),
        in_specs=[pl.BlockSpec((nbe, be), lambda c: (0, 0))],
        out_specs=[pl.BlockSpec((1, nbe, be), lambda c: (c, 0, 0)),
                   pl.BlockSpec((1, nbe, be), lambda c: (c, 0, 0))],
        out_shape=(jax.ShapeDtypeStruct((NC, nbe, be), jnp.int32),
                   jax.ShapeDtypeStruct((NC, nbe, be), jnp.int32)),
    )(dst2d)


def _t1_body(x_ref, w_ref, h_ref):
    h_ref[0] = lax.dot_general(
        x_ref[0], w_ref[0], (((1,), (0,)), ((), ())),
        precision=lax.Precision.HIGHEST, preferred_element_type=_f32)


def _t1(xT, W, bn=2000):
    return pl.pallas_call(
        _t1_body,
        grid=(K, N // bn),
        in_specs=[pl.BlockSpec((1, bn, D), lambda k, i: (k, i, 0)),
                  pl.BlockSpec((1, D, D), lambda k, i: (k, 0, 0))],
        out_specs=pl.BlockSpec((1, bn, D), lambda k, i: (k, i, 0)),
        out_shape=jax.ShapeDtypeStruct((K, N, D), _f32),
    )(xT, W)


def _t2_body(g_ref, ea_ref, we_ref, a_ref, lg_ref, ex_ref):
    ef = lax.dot_general(ea_ref[...], we_ref[...], (((1,), (0,)), ((), ())),
                         precision=lax.Precision.HIGHEST,
                         preferred_element_type=_f32)
    m = g_ref[...] + ef
    l = jnp.where(m >= 0, m, 0.01 * m)
    logit = jnp.sum(l * a_ref[...], axis=1)
    lg_ref[...] = logit[None, None]
    ex_ref[...] = jnp.exp(logit)[None, None]


def _t2(g, edge_attr, We_k, a_k, be=2560):
    nbe = E // be
    return pl.pallas_call(
        _t2_body,
        grid=(nbe,),
        in_specs=[pl.BlockSpec((be, D), lambda i: (i, 0)),
                  pl.BlockSpec((be, ED), lambda i: (i, 0)),
                  pl.BlockSpec((ED, D), lambda i: (0, 0)),
                  pl.BlockSpec((1, D), lambda i: (0, 0))],
        out_specs=[pl.BlockSpec((1, 1, be), lambda i: (i, 0, 0)),
                   pl.BlockSpec((1, 1, be), lambda i: (i, 0, 0))],
        out_shape=(jax.ShapeDtypeStruct((nbe, 1, be), _f32),
                   jax.ShapeDtypeStruct((nbe, 1, be), _f32)),
    )(g, edge_attr, We_k, a_k.reshape(1, D))


_T3BN = 2048


def _t3_body(np_ref, dp_ref, o_ref):
    p = dp_ref[...]
    rep = jnp.broadcast_to(p[:, None, :], (_T3BN // 8, 8, D)).reshape(_T3BN, D)
    lane = jax.lax.broadcasted_iota(jnp.int32, (_T3BN, D), 1)
    sub = jax.lax.broadcasted_iota(jnp.int32, (_T3BN, D), 0)
    sel = ((lane >> 4) == (sub & 7)).astype(_f32)
    den = jnp.sum(rep * sel, axis=1, keepdims=True) * (1.0 / 16.0)
    o_ref[...] = np_ref[...] / (den + 1e-16)


def _t3(nump, denp):
    bn = _T3BN
    return pl.pallas_call(
        _t3_body,
        grid=(NPAD // bn,),
        in_specs=[pl.BlockSpec((bn, D), lambda i: (i, 0)),
                  pl.BlockSpec((bn // 8, D), lambda i: (i, 0))],
        out_specs=pl.BlockSpec((bn, D), lambda i: (i, 0)),
        out_shape=jax.ShapeDtypeStruct((NPAD, D), _f32),
    )(nump, denp)


def kernel(x, edge_attr, edge_index, W, We, a):
    src = edge_index[0]
    dst = edge_index[1]
    xT = jnp.transpose(x, (2, 0, 1))
    h = _t1(xT, W)
    zn = jnp.zeros((NHT, D), _f32)
    dstr, dend = _t0(dst.reshape(E // 128, 128))
    dstr = dstr.reshape(NC * E)
    dend = dend.reshape(NC * E)

    outs = []
    logits = []
    for k in range(K):
        h_k = h[k]
        g = _s1(h_k, src, dst)
        lg, ex = _t2(g, edge_attr, We[k], a[k])
        ex_flat = ex.reshape(E)
        nump, denp = _s2(h_k, src, dstr, dend, ex_flat, zn)
        outs.append(_t3(nump, denp)[:N])
        logits.append(lg.reshape(E))

    node_embeddings = jnp.stack(outs, axis=-1)
    alphas = jnp.stack(logits, axis=-1)
    return (node_embeddings, alphas)
